# baseline (speedup 1.0000x reference)
import functools

import jax
import jax.numpy as jnp
import numpy as np
from jax.experimental import pallas as pl
from jax.experimental.pallas import tpu as pltpu

D = 1024
H = 16
DN = 32
DR = 32
DV = 64
L = 256
E = 8
K = 2
F = 512
THETA = 10000.0
ALPHA = 0.01
EPS = 1e-6


def _rms(x, w):
    return x * jax.lax.rsqrt(jnp.mean(x * x, axis=-1, keepdims=True) + EPS) * w


def _prep_kernel(x_ref, n1_ref, wq_ref, wdkv_ref, kvn_ref, wukv_ref,
                 q_ref, k_ref, v_ref):
    x = x_ref[...]
    h = _rms(x, n1_ref[...])
    q_ref[...] = h @ wq_ref[...]
    ckv = h @ wdkv_ref[...]
    c_kv = ckv[:, :L]
    k_pe = ckv[:, L:L + DR]
    kv = _rms(c_kv, kvn_ref[...]) @ wukv_ref[...]
    k_parts = []
    v_parts = []
    for hh in range(H):
        base = hh * (DN + DV)
        k_parts.append(kv[:, base:base + DN])
        k_parts.append(k_pe)
        v_parts.append(kv[:, base + DN:base + DN + DV])
    k_ref[...] = jnp.concatenate(k_parts, axis=1)
    v_ref[...] = jnp.concatenate(v_parts, axis=1)


def _rope2d(t, pos, inv, rmat):
    ang = pos * inv
    return t * jnp.cos(ang) + (t @ rmat) * jnp.sin(ang)


def _attn_kernel(q_ref, k_ref, v_ref, pq_ref, pk_ref, inv_ref, r_ref, o_ref,
                 *, bq, seq):
    iq = pl.program_id(1)
    inv = inv_ref[...]
    rmat = r_ref[...]
    q2 = q_ref[...]
    k2 = k_ref[...]
    v2 = v_ref[...]
    pq = pq_ref[...]
    pk = pk_ref[...]
    scale = 1.0 / np.sqrt(DN + DR)
    row = iq * bq + jax.lax.broadcasted_iota(jnp.int32, (bq, 1), 0)
    col = jax.lax.broadcasted_iota(jnp.int32, (1, seq), 1)
    causal = col <= row
    outs = []
    for sub in range(2):
        hd = DN + DR
        q = q2[:, sub * hd:(sub + 1) * hd]
        k = k2[:, sub * hd:(sub + 1) * hd]
        qf = jnp.concatenate(
            [q[:, :DN], _rope2d(q[:, DN:], pq, inv, rmat)], axis=1)
        kf = jnp.concatenate(
            [k[:, :DN], _rope2d(k[:, DN:], pk, inv, rmat)], axis=1)
        s = jax.lax.dot_general(qf, kf, (((1,), (1,)), ((), ()))) * scale
        s = jnp.where(causal, s, jnp.float32(-1e9))
        m = jnp.max(s, axis=-1, keepdims=True)
        p = jnp.exp(s - m)
        denom = jnp.sum(p, axis=-1, keepdims=True)
        v = v2[:, sub * DV:(sub + 1) * DV]
        outs.append((p @ v) / denom)
    o_ref[...] = jnp.concatenate(outs, axis=1)


def _moe_kernel(x_ref, a_ref, wo_ref, n2_ref, rw_ref, w1_ref, b1_ref,
                w2_ref, b2_ref, o_ref, aux_ref, x2_s, h2_s, g_s, oh_s, ps_s,
                *, nt, bt, tokens):
    i = pl.program_id(0)
    e = pl.program_id(1)

    @pl.when(e == 0)
    def _():
        x2 = x_ref[...] + a_ref[...] @ wo_ref[...]
        x2_s[...] = x2
        h2 = _rms(x2, n2_ref[...])
        h2_s[...] = h2
        logits = h2 @ rw_ref[...]
        mx = jnp.max(logits, axis=-1, keepdims=True)
        ex = jnp.exp(logits - mx)
        probs = ex / jnp.sum(ex, axis=-1, keepdims=True)
        r0 = jax.lax.broadcasted_iota(jnp.int32, (E, E), 0)
        c0 = jax.lax.broadcasted_iota(jnp.int32, (E, E), 1)
        ut = (r0 <= c0).astype(jnp.float32)
        v1 = jnp.max(probs, axis=-1, keepdims=True)
        m1 = (probs == v1).astype(jnp.float32)
        first1 = jnp.where(m1 @ ut == 1.0, m1, 0.0)
        probs2 = jnp.where(first1 > 0.0, -1.0, probs)
        v2 = jnp.max(probs2, axis=-1, keepdims=True)
        m2 = (probs2 == v2).astype(jnp.float32)
        first2 = jnp.where(m2 @ ut == 1.0, m2, 0.0)
        tot = v1 + v2
        g_s[...] = first1 * (v1 / tot) + first2 * (v2 / tot)

        @pl.when(i == 0)
        def _():
            oh_s[...] = jnp.zeros_like(oh_s)
            ps_s[...] = jnp.zeros_like(ps_s)

        oh_s[...] += jnp.sum(first1 + first2, axis=0, keepdims=True)
        ps_s[...] += jnp.sum(probs, axis=0, keepdims=True)
        o_ref[...] = x2

    h2 = h2_s[...]
    up = h2 @ w1_ref[0] + b1_ref[0]
    act = jax.nn.gelu(up)
    down = act @ w2_ref[0] + b2_ref[0]
    sel = (jax.lax.broadcasted_iota(jnp.int32, (E, 1), 0) == e).astype(
        jnp.float32)
    o_ref[...] += down * (g_s[...] @ sel)

    @pl.when((i == nt - 1) & (e == E - 1))
    def _():
        f = oh_s[...] / (tokens * K)
        pbar = ps_s[...] / tokens
        aux_ref[...] = ALPHA * E * jnp.sum(f * pbar, axis=-1, keepdims=True)


def kernel(x, position_ids, norm1_w, Wq, Wdkv, kv_norm_w, Wukv, Wo,
           norm2_w, router_w, W1, b1, W2, b2):
    B, S, _ = x.shape
    xs = x.reshape(S, D)
    posf = position_ids.astype(jnp.float32).reshape(S, 1)

    rmat = np.zeros((DR, DR), np.float32)
    for ii in range(DR // 2):
        rmat[2 * ii, 2 * ii + 1] = 1.0
        rmat[2 * ii + 1, 2 * ii] = -1.0
    rmat = jnp.asarray(rmat)
    expo = (2.0 * (np.arange(DR) // 2)) / DR
    inv_full = jnp.asarray((THETA ** (-expo)).astype(np.float32)[None, :])

    BT = 256
    q, k, v = pl.pallas_call(
        _prep_kernel,
        grid=(S // BT,),
        in_specs=[
            pl.BlockSpec((BT, D), lambda i: (i, 0)),
            pl.BlockSpec((1, D), lambda i: (0, 0)),
            pl.BlockSpec((D, H * (DN + DR)), lambda i: (0, 0)),
            pl.BlockSpec((D, L + DR), lambda i: (0, 0)),
            pl.BlockSpec((1, L), lambda i: (0, 0)),
            pl.BlockSpec((L, H * (DN + DV)), lambda i: (0, 0)),
        ],
        out_specs=[
            pl.BlockSpec((BT, H * (DN + DR)), lambda i: (i, 0)),
            pl.BlockSpec((BT, H * (DN + DR)), lambda i: (i, 0)),
            pl.BlockSpec((BT, H * DV), lambda i: (i, 0)),
        ],
        out_shape=[
            jax.ShapeDtypeStruct((S, H * (DN + DR)), jnp.float32),
            jax.ShapeDtypeStruct((S, H * (DN + DR)), jnp.float32),
            jax.ShapeDtypeStruct((S, H * DV), jnp.float32),
        ],
    )(xs, norm1_w.reshape(1, D), Wq, Wdkv, kv_norm_w.reshape(1, L), Wukv)

    BQ = 256
    attn = pl.pallas_call(
        functools.partial(_attn_kernel, bq=BQ, seq=S),
        grid=(H // 2, S // BQ),
        in_specs=[
            pl.BlockSpec((BQ, 2 * (DN + DR)), lambda h, i: (i, h)),
            pl.BlockSpec((S, 2 * (DN + DR)), lambda h, i: (0, h)),
            pl.BlockSpec((S, 2 * DV), lambda h, i: (0, h)),
            pl.BlockSpec((BQ, 1), lambda h, i: (i, 0)),
            pl.BlockSpec((S, 1), lambda h, i: (0, 0)),
            pl.BlockSpec((1, DR), lambda h, i: (0, 0)),
            pl.BlockSpec((DR, DR), lambda h, i: (0, 0)),
        ],
        out_specs=pl.BlockSpec((BQ, 2 * DV), lambda h, i: (i, h)),
        out_shape=jax.ShapeDtypeStruct((S, H * DV), jnp.float32),
    )(q, k, v, posf, posf, inv_full, rmat)

    BT2 = 256
    NT = S // BT2
    xo, aux = pl.pallas_call(
        functools.partial(_moe_kernel, nt=NT, bt=BT2, tokens=S),
        grid=(NT, E),
        in_specs=[
            pl.BlockSpec((BT2, D), lambda i, e: (i, 0)),
            pl.BlockSpec((BT2, H * DV), lambda i, e: (i, 0)),
            pl.BlockSpec((H * DV, D), lambda i, e: (0, 0)),
            pl.BlockSpec((1, D), lambda i, e: (0, 0)),
            pl.BlockSpec((D, E), lambda i, e: (0, 0)),
            pl.BlockSpec((1, D, F), lambda i, e: (e, 0, 0)),
            pl.BlockSpec((1, 1, F), lambda i, e: (e, 0, 0)),
            pl.BlockSpec((1, F, D), lambda i, e: (e, 0, 0)),
            pl.BlockSpec((1, 1, D), lambda i, e: (e, 0, 0)),
        ],
        out_specs=[
            pl.BlockSpec((BT2, D), lambda i, e: (i, 0)),
            pl.BlockSpec((1, 1), lambda i, e: (0, 0)),
        ],
        out_shape=[
            jax.ShapeDtypeStruct((S, D), jnp.float32),
            jax.ShapeDtypeStruct((1, 1), jnp.float32),
        ],
        scratch_shapes=[
            pltpu.VMEM((BT2, D), jnp.float32),
            pltpu.VMEM((BT2, D), jnp.float32),
            pltpu.VMEM((BT2, E), jnp.float32),
            pltpu.VMEM((1, E), jnp.float32),
            pltpu.VMEM((1, E), jnp.float32),
        ],
    )(xs, attn, Wo, norm2_w.reshape(1, D), router_w, W1,
      b1.reshape(E, 1, F), W2, b2.reshape(E, 1, D))

    return xo.reshape(B, S, D), aux.reshape(())
